# baseline (speedup 1.0000x reference)
import jax
import jax.numpy as jnp
from jax.experimental import pallas as pl

_ROWS = 8192
_COLS = 128
_BLOCK_ROWS = 8192


def _copy_block(t_ref, o_ref):
    o_ref[...] = t_ref[...]


def kernel(x, pos_table):
    del x
    n_blocks = _ROWS // _BLOCK_ROWS
    return pl.pallas_call(
        _copy_block,
        out_shape=jax.ShapeDtypeStruct((_ROWS, _COLS), pos_table.dtype),
        grid=(n_blocks,),
        in_specs=[pl.BlockSpec((_BLOCK_ROWS, _COLS), lambda i: (i, 0))],
        out_specs=pl.BlockSpec((_BLOCK_ROWS, _COLS), lambda i: (i, 0)),
    )(pos_table)
